# baseline (device time: 114287 ns/iter reference)
import jax
import jax.numpy as jnp
from jax import lax
from jax.experimental import pallas as pl
from jax.experimental.pallas import tpu as pltpu

B, S, D = 2, 512, 2048
H, Dh, Dr = 16, 128, 32
NJ = H // 4
G4 = NJ * Dh
DC_HALF = 128
SCALE = (Dh + Dr) ** -0.5


def _dot(a, b):
    return jnp.dot(a, b, preferred_element_type=jnp.float32)


def _dot_t(a, b):
    return lax.dot_general(
        a, b, (((1,), (1,)), ((), ())), preferred_element_type=jnp.float32
    )


def _kv_body(x_ref, wdkv_ref, wuk_ref, wuv_ref, wq_ref, wqr_ref,
             k_ref, v_ref, wqm_ref, wqrt_ref,
             c_loc, c_rem, wuk_rem, wuv_rem, wuk_send, wuv_send,
             send_sems, recv_sems):
    my_x = lax.axis_index("x")
    my_y = lax.axis_index("y")
    g = 2 * my_x + my_y
    gn = 2 * my_x + (1 - my_y)
    nbr = (my_x, 1 - my_y)

    barrier = pltpu.get_barrier_semaphore()
    pl.semaphore_signal(
        barrier, inc=1, device_id=nbr, device_id_type=pl.DeviceIdType.MESH
    )
    pl.semaphore_wait(barrier, 1)

    xm = x_ref[...].reshape(B * S, D)
    c_loc[...] = _dot(xm, wdkv_ref[...])
    wuk_send[...] = wuk_ref[:, pl.ds(gn * G4, G4)]
    wuv_send[...] = wuv_ref[:, pl.ds(gn * G4, G4)]

    rdmas = []
    for src, dst, i in (
        (c_loc, c_rem, 0),
        (wuk_send, wuk_rem, 1),
        (wuv_send, wuv_rem, 2),
    ):
        rdma = pltpu.make_async_remote_copy(
            src_ref=src,
            dst_ref=dst,
            send_sem=send_sems.at[i],
            recv_sem=recv_sems.at[i],
            device_id=nbr,
            device_id_type=pl.DeviceIdType.MESH,
        )
        rdma.start()
        rdmas.append(rdma)

    wqm_ref[...] = wq_ref[:, pl.ds(g * G4, G4)]
    wqrt_ref[...] = jnp.transpose(
        wqr_ref[:, pl.ds(g * NJ * Dr, NJ * Dr)]
    ).reshape(NJ, Dr, D)
    k_part = _dot(c_loc[...], wuk_ref[:, pl.ds(g * G4, G4)])
    v_part = _dot(c_loc[...], wuv_ref[:, pl.ds(g * G4, G4)])

    for rdma in rdmas:
        rdma.wait()

    k = k_part + _dot(c_rem[...], wuk_rem[...])
    v = v_part + _dot(c_rem[...], wuv_rem[...])
    k_ref[...] = k.reshape(B, S, G4)
    v_ref[...] = v.reshape(B, S, G4)


def _compute_kv(x, Wdkv, Wuk, Wuv, Wq, Wqr):
    return pl.pallas_call(
        _kv_body,
        out_shape=(
            jax.ShapeDtypeStruct((B, S, G4), jnp.float32),
            jax.ShapeDtypeStruct((B, S, G4), jnp.float32),
            jax.ShapeDtypeStruct((D, G4), jnp.float32),
            jax.ShapeDtypeStruct((NJ, Dr, D), jnp.float32),
        ),
        in_specs=[pl.BlockSpec(memory_space=pltpu.VMEM)] * 6,
        out_specs=(pl.BlockSpec(memory_space=pltpu.VMEM),) * 4,
        scratch_shapes=[
            pltpu.VMEM((B * S, DC_HALF), jnp.float32),
            pltpu.VMEM((B * S, DC_HALF), jnp.float32),
            pltpu.VMEM((DC_HALF, G4), jnp.float32),
            pltpu.VMEM((DC_HALF, G4), jnp.float32),
            pltpu.VMEM((DC_HALF, G4), jnp.float32),
            pltpu.VMEM((DC_HALF, G4), jnp.float32),
            pltpu.SemaphoreType.DMA((3,)),
            pltpu.SemaphoreType.DMA((3,)),
        ],
        compiler_params=pltpu.CompilerParams(
            collective_id=0, has_side_effects=True
        ),
    )(x, Wdkv, Wuk, Wuv, Wq, Wqr)


def _bc_body(x_ref, k_ref, v_ref, wqm_ref, wqrt_ref, wkr_ref, wo_ref,
             out_ref, o_scr, oy, ox0, o3, send_sems, recv_sems):
    my_x = lax.axis_index("x")
    my_y = lax.axis_index("y")
    g = 2 * my_x + my_y
    y_nbr = (my_x, 1 - my_y)
    x_nbr = (1 - my_x, my_y)

    barrier = pltpu.get_barrier_semaphore()
    for nbr in (y_nbr, x_nbr):
        pl.semaphore_signal(
            barrier, inc=1, device_id=nbr, device_id_type=pl.DeviceIdType.MESH
        )
    pl.semaphore_wait(barrier, 2)

    def remote_copy(src, dst, i, dev):
        return pltpu.make_async_remote_copy(
            src_ref=src, dst_ref=dst,
            send_sem=send_sems.at[i], recv_sem=recv_sems.at[i],
            device_id=dev, device_id_type=pl.DeviceIdType.MESH,
        )

    r_y = [remote_copy(o_scr.at[b], oy.at[b], b, y_nbr) for b in range(B)]
    r_x = [remote_copy(o_scr.at[b], ox0.at[b], B + b, x_nbr) for b in range(B)]
    r_fwd_x = remote_copy(oy.at[0], o3.at[0], 2 * B, x_nbr)
    r_fwd_y = remote_copy(ox0.at[1], o3.at[1], 2 * B + 1, y_nbr)

    for b in range(B):
        xb = x_ref[b]
        kr = _dot(xb, wkr_ref[...])
        for j in range(NJ):
            hs = slice(j * Dh, (j + 1) * Dh)
            q = _dot(xb, wqm_ref[:, hs])
            qr = _dot_t(xb, wqrt_ref[j])
            scores = (_dot_t(q, k_ref[b, :, hs]) + _dot_t(qr, kr)) * SCALE
            m = jnp.max(scores, axis=-1, keepdims=True)
            p = jnp.exp(scores - m)
            p = p / jnp.sum(p, axis=-1, keepdims=True)
            o_scr[b, :, hs] = _dot(p, v_ref[b, :, hs])
        r_y[b].start()
        r_x[b].start()

    def wo_rows(grp):
        return wo_ref[pl.ds(grp * G4, G4), :]

    for b in range(B):
        out_ref[b] = _dot(o_scr[b], wo_rows(g))

    r_y[0].wait_recv()
    r_fwd_x.start()
    out_ref[0] += _dot(oy[0], wo_rows(g ^ 1))
    r_x[0].wait_recv()
    out_ref[0] += _dot(ox0[0], wo_rows(g ^ 2))
    r_y[1].wait_recv()
    out_ref[1] += _dot(oy[1], wo_rows(g ^ 1))
    r_x[1].wait_recv()
    r_fwd_y.start()
    out_ref[1] += _dot(ox0[1], wo_rows(g ^ 2))

    r_fwd_x.wait_recv()
    out_ref[0] += _dot(o3[0], wo_rows(g ^ 3))
    r_fwd_y.wait_recv()
    out_ref[1] += _dot(o3[1], wo_rows(g ^ 3))

    for r in (*r_y, *r_x, r_fwd_x, r_fwd_y):
        r.wait_send()


def _attn_gather_project(x, K, V, Wq_my, Wqr_t, Wkr, Wo):
    return pl.pallas_call(
        _bc_body,
        out_shape=jax.ShapeDtypeStruct((B, S, D), jnp.float32),
        in_specs=[pl.BlockSpec(memory_space=pltpu.VMEM)] * 7,
        out_specs=pl.BlockSpec(memory_space=pltpu.VMEM),
        scratch_shapes=[
            pltpu.VMEM((B, S, G4), jnp.float32),
            pltpu.VMEM((B, S, G4), jnp.float32),
            pltpu.VMEM((B, S, G4), jnp.float32),
            pltpu.VMEM((B, S, G4), jnp.float32),
            pltpu.SemaphoreType.DMA((6,)),
            pltpu.SemaphoreType.DMA((6,)),
        ],
        compiler_params=pltpu.CompilerParams(
            collective_id=1, has_side_effects=True,
            vmem_limit_bytes=100 * 1024 * 1024,
        ),
    )(x, K, V, Wq_my, Wqr_t, Wkr, Wo)


def kernel(x, Wdkv, Wuk, Wuv, Wq, Wqr, Wkr, Wo):
    K, V, Wq_my, Wqr_t = _compute_kv(x, Wdkv, Wuk, Wuv, Wq, Wqr)
    return _attn_gather_project(x, K, V, Wq_my, Wqr_t, Wkr, Wo)


# device time: 106541 ns/iter; 1.0727x vs baseline; 1.0727x over previous
import jax
import jax.numpy as jnp
from jax import lax
from jax.experimental import pallas as pl
from jax.experimental.pallas import tpu as pltpu

B, S, D = 2, 512, 2048
H, Dh, Dr = 16, 128, 32
NJ = H // 4
G4 = NJ * Dh
DC_HALF = 128
SCALE = (Dh + Dr) ** -0.5


def _dot(a, b):
    return jnp.dot(
        a.astype(jnp.bfloat16), b.astype(jnp.bfloat16),
        preferred_element_type=jnp.float32,
    )


def _dot_t(a, b):
    return lax.dot_general(
        a.astype(jnp.bfloat16), b.astype(jnp.bfloat16),
        (((1,), (1,)), ((), ())),
        preferred_element_type=jnp.float32,
    )


def _kv_body(x_ref, wdkv_ref, wuk_ref, wuv_ref, wqr_ref,
             k_ref, v_ref, wqrt_ref,
             c_loc, c_rem, wuk_rem, wuv_rem, wuk_send, wuv_send,
             send_sems, recv_sems):
    my_x = lax.axis_index("x")
    my_y = lax.axis_index("y")
    g = 2 * my_x + my_y
    gn = 2 * my_x + (1 - my_y)
    nbr = (my_x, 1 - my_y)

    barrier = pltpu.get_barrier_semaphore()
    pl.semaphore_signal(
        barrier, inc=1, device_id=nbr, device_id_type=pl.DeviceIdType.MESH
    )
    pl.semaphore_wait(barrier, 1)

    xm = x_ref[...].reshape(B * S, D)
    c_loc[...] = _dot(xm, wdkv_ref[...])
    wuk_send[...] = wuk_ref[:, pl.ds(gn * G4, G4)]
    wuv_send[...] = wuv_ref[:, pl.ds(gn * G4, G4)]

    rdmas = []
    for src, dst, i in (
        (c_loc, c_rem, 0),
        (wuk_send, wuk_rem, 1),
        (wuv_send, wuv_rem, 2),
    ):
        rdma = pltpu.make_async_remote_copy(
            src_ref=src,
            dst_ref=dst,
            send_sem=send_sems.at[i],
            recv_sem=recv_sems.at[i],
            device_id=nbr,
            device_id_type=pl.DeviceIdType.MESH,
        )
        rdma.start()
        rdmas.append(rdma)

    wqrt_ref[...] = jnp.transpose(
        wqr_ref[:, pl.ds(g * NJ * Dr, NJ * Dr)]
    ).reshape(NJ, Dr, D)
    k_part = _dot(c_loc[...], wuk_ref[:, pl.ds(g * G4, G4)])
    v_part = _dot(c_loc[...], wuv_ref[:, pl.ds(g * G4, G4)])

    for rdma in rdmas:
        rdma.wait()

    k = k_part + _dot(c_rem[...], wuk_rem[...])
    v = v_part + _dot(c_rem[...], wuv_rem[...])
    k_ref[...] = k.reshape(B, S, G4)
    v_ref[...] = v.reshape(B, S, G4)


def _compute_kv(x, Wdkv, Wuk, Wuv, Wqr):
    return pl.pallas_call(
        _kv_body,
        out_shape=(
            jax.ShapeDtypeStruct((B, S, G4), jnp.float32),
            jax.ShapeDtypeStruct((B, S, G4), jnp.float32),
            jax.ShapeDtypeStruct((NJ, Dr, D), jnp.float32),
        ),
        in_specs=[pl.BlockSpec(memory_space=pltpu.VMEM)] * 5,
        out_specs=(pl.BlockSpec(memory_space=pltpu.VMEM),) * 3,
        scratch_shapes=[
            pltpu.VMEM((B * S, DC_HALF), jnp.float32),
            pltpu.VMEM((B * S, DC_HALF), jnp.float32),
            pltpu.VMEM((DC_HALF, G4), jnp.float32),
            pltpu.VMEM((DC_HALF, G4), jnp.float32),
            pltpu.VMEM((DC_HALF, G4), jnp.float32),
            pltpu.VMEM((DC_HALF, G4), jnp.float32),
            pltpu.SemaphoreType.DMA((3,)),
            pltpu.SemaphoreType.DMA((3,)),
        ],
        compiler_params=pltpu.CompilerParams(
            collective_id=0, has_side_effects=True
        ),
    )(x, Wdkv, Wuk, Wuv, Wqr)


def _attn_body(g_ref, x_ref, k_ref, v_ref, wq_ref, wqrt_ref, wkr_ref,
               o_ref, kr_scratch):
    del g_ref
    j = pl.program_id(1)
    xb = x_ref[0]

    @pl.when(j == 0)
    def _():
        kr_scratch[...] = _dot(xb, wkr_ref[...])

    q = _dot(xb, wq_ref[...])
    qr = _dot_t(xb, wqrt_ref[0])

    scores = (_dot_t(q, k_ref[0]) + _dot_t(qr, kr_scratch[...])) * SCALE
    m = jnp.max(scores, axis=-1, keepdims=True)
    p = jnp.exp(scores - m)
    p = p / jnp.sum(p, axis=-1, keepdims=True)
    o_ref[0] = _dot(p, v_ref[0])


def _attention(g, x, K, V, Wq, Wqr_t, Wkr):
    grid_spec = pltpu.PrefetchScalarGridSpec(
        num_scalar_prefetch=1,
        grid=(B, NJ),
        in_specs=[
            pl.BlockSpec((1, S, D), lambda b, j, g: (b, 0, 0)),
            pl.BlockSpec((1, S, Dh), lambda b, j, g: (b, 0, j)),
            pl.BlockSpec((1, S, Dh), lambda b, j, g: (b, 0, j)),
            pl.BlockSpec((D, Dh), lambda b, j, g: (0, g[0] * NJ + j)),
            pl.BlockSpec((1, Dr, D), lambda b, j, g: (j, 0, 0)),
            pl.BlockSpec((D, Dr), lambda b, j, g: (0, 0)),
        ],
        out_specs=pl.BlockSpec((1, S, Dh), lambda b, j, g: (b, 0, j)),
        scratch_shapes=[pltpu.VMEM((S, Dr), jnp.float32)],
    )
    return pl.pallas_call(
        _attn_body,
        grid_spec=grid_spec,
        out_shape=jax.ShapeDtypeStruct((B, S, G4), jnp.float32),
        compiler_params=pltpu.CompilerParams(
            dimension_semantics=("arbitrary", "arbitrary"),
        ),
    )(g, x, K, V, Wq, Wqr_t, Wkr)


def _gather_body(o_ref, wo_ref, out_ref, oy, ox0, o3, send_sems, recv_sems):
    my_x = lax.axis_index("x")
    my_y = lax.axis_index("y")
    g = 2 * my_x + my_y
    y_nbr = (my_x, 1 - my_y)
    x_nbr = (1 - my_x, my_y)

    barrier = pltpu.get_barrier_semaphore()
    for nbr in (y_nbr, x_nbr):
        pl.semaphore_signal(
            barrier, inc=1, device_id=nbr, device_id_type=pl.DeviceIdType.MESH
        )
    pl.semaphore_wait(barrier, 2)

    def remote_copy(src, dst, i, dev):
        return pltpu.make_async_remote_copy(
            src_ref=src, dst_ref=dst,
            send_sem=send_sems.at[i], recv_sem=recv_sems.at[i],
            device_id=dev, device_id_type=pl.DeviceIdType.MESH,
        )

    r_y = remote_copy(o_ref, oy, 0, y_nbr)
    r_x0 = remote_copy(o_ref, ox0, 1, x_nbr)
    r_y.start()
    r_x0.start()

    def wo_rows(grp):
        return wo_ref[pl.ds(grp * G4, G4), :]

    out_ref[...] = _dot(o_ref[...].reshape(B * S, G4), wo_rows(g)).reshape(B, S, D)

    r_y.wait_recv()
    r3 = remote_copy(oy.at[0], o3.at[0], 2, x_nbr)
    r3.start()
    out_ref[...] += _dot(oy[...].reshape(B * S, G4), wo_rows(g ^ 1)).reshape(B, S, D)

    r_x0.wait_recv()
    r4 = remote_copy(ox0.at[1], o3.at[1], 3, y_nbr)
    r4.start()
    out_ref[...] += _dot(ox0[...].reshape(B * S, G4), wo_rows(g ^ 2)).reshape(B, S, D)

    r3.wait_recv()
    out_ref[0] += _dot(o3[0], wo_rows(g ^ 3))
    r4.wait_recv()
    out_ref[1] += _dot(o3[1], wo_rows(g ^ 3))

    for r in (r_y, r_x0, r3, r4):
        r.wait_send()


def _gather_project(O, Wo):
    return pl.pallas_call(
        _gather_body,
        out_shape=jax.ShapeDtypeStruct((B, S, D), jnp.float32),
        in_specs=[pl.BlockSpec(memory_space=pltpu.VMEM)] * 2,
        out_specs=pl.BlockSpec(memory_space=pltpu.VMEM),
        scratch_shapes=[
            pltpu.VMEM((B, S, G4), jnp.float32),
            pltpu.VMEM((B, S, G4), jnp.float32),
            pltpu.VMEM((B, S, G4), jnp.float32),
            pltpu.SemaphoreType.DMA((4,)),
            pltpu.SemaphoreType.DMA((4,)),
        ],
        compiler_params=pltpu.CompilerParams(
            collective_id=1, has_side_effects=True
        ),
    )(O, Wo)


def kernel(x, Wdkv, Wuk, Wuv, Wq, Wqr, Wkr, Wo):
    K, V, Wqr_t = _compute_kv(x, Wdkv, Wuk, Wuv, Wqr)
    g = (2 * lax.axis_index("x") + lax.axis_index("y")).astype(jnp.int32)
    O = _attention(jnp.reshape(g, (1,)), x, K, V, Wq, Wqr_t, Wkr)
    return _gather_project(O, Wo)


# device time: 83399 ns/iter; 1.3704x vs baseline; 1.2775x over previous
import jax
import jax.numpy as jnp
from jax import lax
from jax.experimental import pallas as pl
from jax.experimental.pallas import tpu as pltpu

B, S, D = 2, 512, 2048
H, Dh, Dr = 16, 128, 32
NJ = H // 4
G4 = NJ * Dh
DC_HALF = 128
SCALE = (Dh + Dr) ** -0.5


def _dot(a, b):
    return jnp.dot(
        a.astype(jnp.bfloat16), b.astype(jnp.bfloat16),
        preferred_element_type=jnp.float32,
    )


def _dot_t(a, b):
    return lax.dot_general(
        a.astype(jnp.bfloat16), b.astype(jnp.bfloat16),
        (((1,), (1,)), ((), ())),
        preferred_element_type=jnp.float32,
    )


def _kv_body(x_ref, wdkv_ref, wuk_ref, wuv_ref, wqr_ref,
             k_ref, v_ref, wqrt_ref,
             c_loc, c_rem, wuk_rem, wuv_rem, wuk_send, wuv_send,
             send_sems, recv_sems):
    my_x = lax.axis_index("x")
    my_y = lax.axis_index("y")
    g = 2 * my_x + my_y
    gn = 2 * my_x + (1 - my_y)
    nbr = (my_x, 1 - my_y)

    barrier = pltpu.get_barrier_semaphore()
    pl.semaphore_signal(
        barrier, inc=1, device_id=nbr, device_id_type=pl.DeviceIdType.MESH
    )
    pl.semaphore_wait(barrier, 1)

    xm = x_ref[...].reshape(B * S, D)
    c_loc[...] = _dot(xm, wdkv_ref[...]).astype(jnp.bfloat16)
    wuk_send[...] = wuk_ref[:, pl.ds(gn * G4, G4)].astype(jnp.bfloat16)
    wuv_send[...] = wuv_ref[:, pl.ds(gn * G4, G4)].astype(jnp.bfloat16)

    rdmas = []
    for src, dst, i in (
        (c_loc, c_rem, 0),
        (wuk_send, wuk_rem, 1),
        (wuv_send, wuv_rem, 2),
    ):
        rdma = pltpu.make_async_remote_copy(
            src_ref=src,
            dst_ref=dst,
            send_sem=send_sems.at[i],
            recv_sem=recv_sems.at[i],
            device_id=nbr,
            device_id_type=pl.DeviceIdType.MESH,
        )
        rdma.start()
        rdmas.append(rdma)

    wqrt_ref[...] = jnp.transpose(
        wqr_ref[:, pl.ds(g * NJ * Dr, NJ * Dr)]
    ).reshape(NJ, Dr, D)
    k_part = _dot(c_loc[...], wuk_ref[:, pl.ds(g * G4, G4)])
    v_part = _dot(c_loc[...], wuv_ref[:, pl.ds(g * G4, G4)])

    for rdma in rdmas:
        rdma.wait()

    k = k_part + _dot(c_rem[...], wuk_rem[...])
    v = v_part + _dot(c_rem[...], wuv_rem[...])
    k_ref[...] = k.reshape(B, S, G4).astype(jnp.bfloat16)
    v_ref[...] = v.reshape(B, S, G4).astype(jnp.bfloat16)


def _compute_kv(x, Wdkv, Wuk, Wuv, Wqr):
    return pl.pallas_call(
        _kv_body,
        out_shape=(
            jax.ShapeDtypeStruct((B, S, G4), jnp.bfloat16),
            jax.ShapeDtypeStruct((B, S, G4), jnp.bfloat16),
            jax.ShapeDtypeStruct((NJ, Dr, D), jnp.float32),
        ),
        in_specs=[pl.BlockSpec(memory_space=pltpu.VMEM)] * 5,
        out_specs=(pl.BlockSpec(memory_space=pltpu.VMEM),) * 3,
        scratch_shapes=[
            pltpu.VMEM((B * S, DC_HALF), jnp.bfloat16),
            pltpu.VMEM((B * S, DC_HALF), jnp.bfloat16),
            pltpu.VMEM((DC_HALF, G4), jnp.bfloat16),
            pltpu.VMEM((DC_HALF, G4), jnp.bfloat16),
            pltpu.VMEM((DC_HALF, G4), jnp.bfloat16),
            pltpu.VMEM((DC_HALF, G4), jnp.bfloat16),
            pltpu.SemaphoreType.DMA((3,)),
            pltpu.SemaphoreType.DMA((3,)),
        ],
        compiler_params=pltpu.CompilerParams(
            collective_id=0, has_side_effects=True
        ),
    )(x, Wdkv, Wuk, Wuv, Wqr)


def _attn_body(g_ref, x_ref, k_ref, v_ref, wq_ref, wqrt_ref, wkr_ref,
               o_ref, kr_scratch):
    del g_ref
    j = pl.program_id(1)
    xb = x_ref[0]

    @pl.when(j == 0)
    def _():
        kr_scratch[...] = _dot(xb, wkr_ref[...])

    q = _dot(xb, wq_ref[...])
    qr = _dot_t(xb, wqrt_ref[0])

    scores = (_dot_t(q, k_ref[0]) + _dot_t(qr, kr_scratch[...])) * SCALE
    m = jnp.max(scores, axis=-1, keepdims=True)
    p = jnp.exp(scores - m)
    p = p / jnp.sum(p, axis=-1, keepdims=True)
    o_ref[0] = _dot(p, v_ref[0]).astype(jnp.bfloat16)


def _attention(g, x, K, V, Wq, Wqr_t, Wkr):
    grid_spec = pltpu.PrefetchScalarGridSpec(
        num_scalar_prefetch=1,
        grid=(B, NJ),
        in_specs=[
            pl.BlockSpec((1, S, D), lambda b, j, g: (b, 0, 0)),
            pl.BlockSpec((1, S, Dh), lambda b, j, g: (b, 0, j)),
            pl.BlockSpec((1, S, Dh), lambda b, j, g: (b, 0, j)),
            pl.BlockSpec((D, Dh), lambda b, j, g: (0, g[0] * NJ + j)),
            pl.BlockSpec((1, Dr, D), lambda b, j, g: (j, 0, 0)),
            pl.BlockSpec((D, Dr), lambda b, j, g: (0, 0)),
        ],
        out_specs=pl.BlockSpec((1, S, Dh), lambda b, j, g: (b, 0, j)),
        scratch_shapes=[pltpu.VMEM((S, Dr), jnp.float32)],
    )
    return pl.pallas_call(
        _attn_body,
        grid_spec=grid_spec,
        out_shape=jax.ShapeDtypeStruct((B, S, G4), jnp.bfloat16),
        compiler_params=pltpu.CompilerParams(
            dimension_semantics=("arbitrary", "arbitrary"),
        ),
    )(g, x, K, V, Wq, Wqr_t, Wkr)


def _gather_body(o_ref, wo_ref, out_ref, oy, ox0, o3, send_sems, recv_sems):
    my_x = lax.axis_index("x")
    my_y = lax.axis_index("y")
    g = 2 * my_x + my_y
    y_nbr = (my_x, 1 - my_y)
    x_nbr = (1 - my_x, my_y)

    barrier = pltpu.get_barrier_semaphore()
    for nbr in (y_nbr, x_nbr):
        pl.semaphore_signal(
            barrier, inc=1, device_id=nbr, device_id_type=pl.DeviceIdType.MESH
        )
    pl.semaphore_wait(barrier, 2)

    def remote_copy(src, dst, i, dev):
        return pltpu.make_async_remote_copy(
            src_ref=src, dst_ref=dst,
            send_sem=send_sems.at[i], recv_sem=recv_sems.at[i],
            device_id=dev, device_id_type=pl.DeviceIdType.MESH,
        )

    r_y = remote_copy(o_ref, oy, 0, y_nbr)
    r_x0 = remote_copy(o_ref, ox0, 1, x_nbr)
    r_y.start()
    r_x0.start()

    def wo_rows(grp):
        return wo_ref[pl.ds(grp * G4, G4), :]

    out_ref[...] = _dot(o_ref[...].reshape(B * S, G4), wo_rows(g)).reshape(B, S, D)

    r_y.wait_recv()
    r3 = remote_copy(oy.at[0], o3.at[0], 2, x_nbr)
    r3.start()
    out_ref[...] += _dot(oy[...].reshape(B * S, G4), wo_rows(g ^ 1)).reshape(B, S, D)

    r_x0.wait_recv()
    r4 = remote_copy(ox0.at[1], o3.at[1], 3, y_nbr)
    r4.start()
    out_ref[...] += _dot(ox0[...].reshape(B * S, G4), wo_rows(g ^ 2)).reshape(B, S, D)

    r3.wait_recv()
    out_ref[0] += _dot(o3[0], wo_rows(g ^ 3))
    r4.wait_recv()
    out_ref[1] += _dot(o3[1], wo_rows(g ^ 3))

    for r in (r_y, r_x0, r3, r4):
        r.wait_send()


def _gather_project(O, Wo):
    return pl.pallas_call(
        _gather_body,
        out_shape=jax.ShapeDtypeStruct((B, S, D), jnp.float32),
        in_specs=[pl.BlockSpec(memory_space=pltpu.VMEM)] * 2,
        out_specs=pl.BlockSpec(memory_space=pltpu.VMEM),
        scratch_shapes=[
            pltpu.VMEM((B, S, G4), jnp.bfloat16),
            pltpu.VMEM((B, S, G4), jnp.bfloat16),
            pltpu.VMEM((B, S, G4), jnp.bfloat16),
            pltpu.SemaphoreType.DMA((4,)),
            pltpu.SemaphoreType.DMA((4,)),
        ],
        compiler_params=pltpu.CompilerParams(
            collective_id=1, has_side_effects=True
        ),
    )(O, Wo)


def kernel(x, Wdkv, Wuk, Wuv, Wq, Wqr, Wkr, Wo):
    K, V, Wqr_t = _compute_kv(x, Wdkv, Wuk, Wuv, Wqr)
    g = (2 * lax.axis_index("x") + lax.axis_index("y")).astype(jnp.int32)
    O = _attention(jnp.reshape(g, (1,)), x, K, V, Wq, Wqr_t, Wkr)
    return _gather_project(O, Wo)


# device time: 72314 ns/iter; 1.5804x vs baseline; 1.1533x over previous
import jax
import jax.numpy as jnp
from jax import lax
from jax.experimental import pallas as pl
from jax.experimental.pallas import tpu as pltpu

B, S, D = 2, 512, 2048
H, Dh, Dr = 16, 128, 32
NJ = H // 4
G4 = NJ * Dh
DC_HALF = 128
SCALE = (Dh + Dr) ** -0.5


def _dot(a, b):
    return jnp.dot(
        a.astype(jnp.bfloat16), b.astype(jnp.bfloat16),
        preferred_element_type=jnp.float32,
    )


def _dot_t(a, b):
    return lax.dot_general(
        a.astype(jnp.bfloat16), b.astype(jnp.bfloat16),
        (((1,), (1,)), ((), ())),
        preferred_element_type=jnp.float32,
    )


def _kv_body(x_ref, wdkv_ref, wuk_ref, wuv_ref, wqr_ref,
             k_ref, v_ref, wqrt_ref,
             c_loc, c_rem, wuk_rem, wuv_rem, wuk_send, wuv_send,
             send_sems, recv_sems):
    my_x = lax.axis_index("x")
    my_y = lax.axis_index("y")
    g = 2 * my_x + my_y
    gn = 2 * my_x + (1 - my_y)
    nbr = (my_x, 1 - my_y)

    barrier = pltpu.get_barrier_semaphore()
    pl.semaphore_signal(
        barrier, inc=1, device_id=nbr, device_id_type=pl.DeviceIdType.MESH
    )
    pl.semaphore_wait(barrier, 1)

    xm = x_ref[...].reshape(B * S, D)
    c_loc[...] = _dot(xm, wdkv_ref[...]).astype(jnp.bfloat16)
    wuk_send[...] = wuk_ref[:, pl.ds(gn * G4, G4)].astype(jnp.bfloat16)
    wuv_send[...] = wuv_ref[:, pl.ds(gn * G4, G4)].astype(jnp.bfloat16)

    rdmas = []
    for src, dst, i in (
        (c_loc, c_rem, 0),
        (wuk_send, wuk_rem, 1),
        (wuv_send, wuv_rem, 2),
    ):
        rdma = pltpu.make_async_remote_copy(
            src_ref=src,
            dst_ref=dst,
            send_sem=send_sems.at[i],
            recv_sem=recv_sems.at[i],
            device_id=nbr,
            device_id_type=pl.DeviceIdType.MESH,
        )
        rdma.start()
        rdmas.append(rdma)

    wqrt_ref[...] = jnp.transpose(wqr_ref[:, pl.ds(g * NJ * Dr, NJ * Dr)])
    k_part = _dot(c_loc[...], wuk_ref[:, pl.ds(g * G4, G4)])
    v_part = _dot(c_loc[...], wuv_ref[:, pl.ds(g * G4, G4)])

    for rdma in rdmas:
        rdma.wait()

    k = k_part + _dot(c_rem[...], wuk_rem[...])
    v = v_part + _dot(c_rem[...], wuv_rem[...])
    k_ref[...] = k.reshape(B, S, G4).astype(jnp.bfloat16)
    v_ref[...] = v.reshape(B, S, G4).astype(jnp.bfloat16)


def _compute_kv(x, Wdkv, Wuk, Wuv, Wqr):
    return pl.pallas_call(
        _kv_body,
        out_shape=(
            jax.ShapeDtypeStruct((B, S, G4), jnp.bfloat16),
            jax.ShapeDtypeStruct((B, S, G4), jnp.bfloat16),
            jax.ShapeDtypeStruct((NJ * Dr, D), jnp.float32),
        ),
        in_specs=[pl.BlockSpec(memory_space=pltpu.VMEM)] * 5,
        out_specs=(pl.BlockSpec(memory_space=pltpu.VMEM),) * 3,
        scratch_shapes=[
            pltpu.VMEM((B * S, DC_HALF), jnp.bfloat16),
            pltpu.VMEM((B * S, DC_HALF), jnp.bfloat16),
            pltpu.VMEM((DC_HALF, G4), jnp.bfloat16),
            pltpu.VMEM((DC_HALF, G4), jnp.bfloat16),
            pltpu.VMEM((DC_HALF, G4), jnp.bfloat16),
            pltpu.VMEM((DC_HALF, G4), jnp.bfloat16),
            pltpu.SemaphoreType.DMA((3,)),
            pltpu.SemaphoreType.DMA((3,)),
        ],
        compiler_params=pltpu.CompilerParams(
            collective_id=0, has_side_effects=True
        ),
    )(x, Wdkv, Wuk, Wuv, Wqr)


def _attn_body(g_ref, x_ref, k_ref, v_ref, wq_ref, wqrt_ref, wkr_ref, o_ref):
    del g_ref
    xb = x_ref[0]

    kr = _dot(xb, wkr_ref[...])
    q_all = _dot(xb, wq_ref[...])
    qr_all = _dot_t(xb, wqrt_ref[...])

    for j in range(NJ):
        hs = slice(j * Dh, (j + 1) * Dh)
        rs = slice(j * Dr, (j + 1) * Dr)
        scores = (
            _dot_t(q_all[:, hs], k_ref[0, :, hs])
            + _dot_t(qr_all[:, rs], kr)
        ) * SCALE
        m = jnp.max(scores, axis=-1, keepdims=True)
        p = jnp.exp(scores - m)
        p = p / jnp.sum(p, axis=-1, keepdims=True)
        o_ref[0, :, hs] = _dot(p, v_ref[0, :, hs]).astype(jnp.bfloat16)


def _attention(g, x, K, V, Wq, Wqr_t, Wkr):
    grid_spec = pltpu.PrefetchScalarGridSpec(
        num_scalar_prefetch=1,
        grid=(B,),
        in_specs=[
            pl.BlockSpec((1, S, D), lambda b, g: (b, 0, 0)),
            pl.BlockSpec((1, S, G4), lambda b, g: (b, 0, 0)),
            pl.BlockSpec((1, S, G4), lambda b, g: (b, 0, 0)),
            pl.BlockSpec((D, G4), lambda b, g: (0, g[0])),
            pl.BlockSpec((NJ * Dr, D), lambda b, g: (0, 0)),
            pl.BlockSpec((D, Dr), lambda b, g: (0, 0)),
        ],
        out_specs=pl.BlockSpec((1, S, G4), lambda b, g: (b, 0, 0)),
    )
    return pl.pallas_call(
        _attn_body,
        grid_spec=grid_spec,
        out_shape=jax.ShapeDtypeStruct((B, S, G4), jnp.bfloat16),
        compiler_params=pltpu.CompilerParams(
            dimension_semantics=("arbitrary",),
        ),
    )(g, x, K, V, Wq, Wqr_t, Wkr)


def _gather_body(o_ref, wo_ref, out_ref, oy, ox0, o3, send_sems, recv_sems):
    my_x = lax.axis_index("x")
    my_y = lax.axis_index("y")
    g = 2 * my_x + my_y
    y_nbr = (my_x, 1 - my_y)
    x_nbr = (1 - my_x, my_y)

    barrier = pltpu.get_barrier_semaphore()
    for nbr in (y_nbr, x_nbr):
        pl.semaphore_signal(
            barrier, inc=1, device_id=nbr, device_id_type=pl.DeviceIdType.MESH
        )
    pl.semaphore_wait(barrier, 2)

    def remote_copy(src, dst, i, dev):
        return pltpu.make_async_remote_copy(
            src_ref=src, dst_ref=dst,
            send_sem=send_sems.at[i], recv_sem=recv_sems.at[i],
            device_id=dev, device_id_type=pl.DeviceIdType.MESH,
        )

    r_y = remote_copy(o_ref, oy, 0, y_nbr)
    r_x0 = remote_copy(o_ref, ox0, 1, x_nbr)
    r_y.start()
    r_x0.start()

    def wo_rows(grp):
        return wo_ref[pl.ds(grp * G4, G4), :]

    out_ref[...] = _dot(o_ref[...].reshape(B * S, G4), wo_rows(g)).reshape(B, S, D)

    r_y.wait_recv()
    r3 = remote_copy(oy.at[0], o3.at[0], 2, x_nbr)
    r3.start()
    out_ref[...] += _dot(oy[...].reshape(B * S, G4), wo_rows(g ^ 1)).reshape(B, S, D)

    r_x0.wait_recv()
    r4 = remote_copy(ox0.at[1], o3.at[1], 3, y_nbr)
    r4.start()
    out_ref[...] += _dot(ox0[...].reshape(B * S, G4), wo_rows(g ^ 2)).reshape(B, S, D)

    r3.wait_recv()
    out_ref[0] += _dot(o3[0], wo_rows(g ^ 3))
    r4.wait_recv()
    out_ref[1] += _dot(o3[1], wo_rows(g ^ 3))

    for r in (r_y, r_x0, r3, r4):
        r.wait_send()


def _gather_project(O, Wo):
    return pl.pallas_call(
        _gather_body,
        out_shape=jax.ShapeDtypeStruct((B, S, D), jnp.float32),
        in_specs=[pl.BlockSpec(memory_space=pltpu.VMEM)] * 2,
        out_specs=pl.BlockSpec(memory_space=pltpu.VMEM),
        scratch_shapes=[
            pltpu.VMEM((B, S, G4), jnp.bfloat16),
            pltpu.VMEM((B, S, G4), jnp.bfloat16),
            pltpu.VMEM((B, S, G4), jnp.bfloat16),
            pltpu.SemaphoreType.DMA((4,)),
            pltpu.SemaphoreType.DMA((4,)),
        ],
        compiler_params=pltpu.CompilerParams(
            collective_id=1, has_side_effects=True
        ),
    )(O, Wo)


def kernel(x, Wdkv, Wuk, Wuv, Wq, Wqr, Wkr, Wo):
    K, V, Wqr_t = _compute_kv(x, Wdkv, Wuk, Wuv, Wqr)
    g = (2 * lax.axis_index("x") + lax.axis_index("y")).astype(jnp.int32)
    O = _attention(jnp.reshape(g, (1,)), x, K, V, Wq, Wqr_t, Wkr)
    return _gather_project(O, Wo)
